# baseline (speedup 1.0000x reference)
import jax
import jax.numpy as jnp
from jax import lax
from jax.experimental import pallas as pl
from jax.experimental.pallas import tpu as pltpu
from jax.experimental.pallas import tpu_sc as plsc

_K = 8
_LW = 128
_NC = 2
_NS = 16
_L = 16


def _proj_body(x_ref, wq_ref, bq_ref, q_ref):
    q = lax.dot_general(
        x_ref[...], wq_ref[...],
        (((1,), (1,)), ((), ())),
        preferred_element_type=jnp.float32,
    )
    q_ref[...] = q + bq_ref[...]


def _topk_body(q_ref, m0_ref, m1_ref, m2_ref, m3_ref, pos_ref, wts_ref):
    q = q_ref[0]
    r_sz = q.shape[0]

    qn = q / (jnp.sqrt(jnp.sum(q * q, axis=-1, keepdims=True)) + 1e-8)

    sims = []
    m_sz = 0
    for ref in (m0_ref, m1_ref, m2_ref, m3_ref):
        mem = ref[0, 0]
        qm, w_dim = mem.shape
        m_sz += qm
        mem3 = mem.reshape(qm // _LW, _LW, w_dim)
        ssq = jnp.sum(mem3 * mem3, axis=-1)
        inv = (1.0 / (jnp.sqrt(ssq) + 1e-8)).reshape(1, qm)
        sims.append(lax.dot_general(
            qn, mem,
            (((1,), (1,)), ((), ())),
            preferred_element_type=jnp.float32,
        ) * inv)

    q_len = sims[0].shape[1]
    n_ch = q_len // _LW
    iota_l = lax.broadcasted_iota(jnp.int32, (r_sz, _LW), 1)
    neg = jnp.float32(-3.0e38)
    big = jnp.int32(m_sz)

    g1 = sims[0][:, 0:_LW]
    i1 = iota_l
    g2 = jnp.full((r_sz, _LW), neg, jnp.float32)
    i2 = jnp.full((r_sz, _LW), big, jnp.int32)
    for p, s in enumerate(sims):
        for j in range(n_ch):
            if p == 0 and j == 0:
                continue
            v = s[:, j * _LW:(j + 1) * _LW]
            col = iota_l + (p * q_len + j * _LW)
            c1 = v > g1
            c2 = v > g2
            g2 = jnp.where(c1, g1, jnp.where(c2, v, g2))
            i2 = jnp.where(c1, i1, jnp.where(c2, col, i2))
            g1 = jnp.where(c1, v, g1)
            i1 = jnp.where(c1, col, i1)

    def _rescan(excluded):
        rg1 = jnp.full((r_sz, _LW), neg, jnp.float32)
        ri1 = jnp.full((r_sz, _LW), big, jnp.int32)
        rg2 = jnp.full((r_sz, _LW), neg, jnp.float32)
        ri2 = jnp.full((r_sz, _LW), big, jnp.int32)
        for pp, ss in enumerate(sims):
            for jj in range(n_ch):
                vv = ss[:, jj * _LW:(jj + 1) * _LW]
                ccol = iota_l + (pp * q_len + jj * _LW)
                excl = jnp.zeros((r_sz, _LW), jnp.bool_)
                for e in excluded:
                    excl = excl | (ccol == e)
                vv = jnp.where(excl, neg, vv)
                cc1 = vv > rg1
                cc2 = vv > rg2
                rg2 = jnp.where(cc1, rg1, jnp.where(cc2, vv, rg2))
                ri2 = jnp.where(cc1, ri1, jnp.where(cc2, ccol, ri2))
                rg1 = jnp.where(cc1, vv, rg1)
                ri1 = jnp.where(cc1, ccol, ri1)
        return rg1, ri1, rg2, ri2

    denom = jnp.zeros((r_sz, 1), dtype=jnp.float32)
    v0 = None
    cols, ws = [], []
    for k in range(_K):
        vmax = jnp.max(g1, axis=1, keepdims=True)
        csel = jnp.min(jnp.where(g1 == vmax, i1, big),
                       axis=1, keepdims=True)
        cols.append(csel)
        if v0 is None:
            v0 = vmax
        w = jnp.exp(vmax - v0)
        ws.append(w)
        denom = denom + w
        hit = i1 == csel
        third = hit & (i2 == big)
        g1 = jnp.where(hit, g2, g1)
        i1 = jnp.where(hit, i2, i1)
        g2 = jnp.where(hit, neg, g2)
        i2 = jnp.where(hit, big, i2)
        if k < _K - 1:
            need = jnp.any(third)
            g1, i1, g2, i2 = lax.cond(
                need,
                lambda c=tuple(cols): _rescan(c),
                lambda: (g1, i1, g2, i2),
            )

    pos_ref[0] = jnp.concatenate(cols, axis=1)
    wts_ref[0] = jnp.concatenate(ws, axis=1) / denom


def _sc_read_body(mem_ref, pos_ref, wts_ref, out_ref,
                  idxv, wvx, rows, acc, sem):
    cid = lax.axis_index("c")
    sid = lax.axis_index("s")
    b = sid * _NC + cid
    m_sz = mem_ref.shape[0] // (_NC * _NS)
    rk = idxv.shape[0]

    pltpu.sync_copy(pos_ref.at[b], idxv)
    pltpu.sync_copy(wts_ref.at[b], wvx)
    for j in range(rk // _L):
        sl = pl.ds(j * _L, _L)
        idxv[sl] = idxv[sl] + b * m_sz
    pltpu.async_copy(mem_ref.at[idxv], rows, sem).wait()

    r_sz, w_sz = acc.shape
    for r in range(r_sz):
        accs = [jnp.zeros((_L,), jnp.float32) for _ in range(w_sz // _L)]
        for k in range(_K):
            wspl = wvx[r * _K + k]
            for c in range(w_sz // _L):
                accs[c] = accs[c] + wspl * rows[r * _K + k, pl.ds(c * _L, _L)]
        for c in range(w_sz // _L):
            acc[r, pl.ds(c * _L, _L)] = accs[c]
    pltpu.sync_copy(acc, out_ref.at[b])


def kernel(x, memory, W_q, b_q):
    b, m, w = memory.shape
    wr = W_q.shape[0]
    r = wr // w

    q = pl.pallas_call(
        _proj_body,
        out_shape=jax.ShapeDtypeStruct((b, wr), jnp.float32),
    )(x, W_q, b_q.reshape(1, wr))
    q3 = q.reshape(b, r, w)

    n_q = 4
    mq = memory.reshape(b, n_q, m // n_q, w)
    read_positions, weights = pl.pallas_call(
        _topk_body,
        grid=(b,),
        in_specs=[
            pl.BlockSpec((1, r, w), lambda i: (i, 0, 0)),
        ] + [
            pl.BlockSpec((1, 1, m // n_q, w), lambda i, p=p: (i, p, 0, 0))
            for p in range(n_q)
        ],
        out_specs=[
            pl.BlockSpec((1, r, _K), lambda i: (i, 0, 0)),
            pl.BlockSpec((1, r, _K), lambda i: (i, 0, 0)),
        ],
        out_shape=[
            jax.ShapeDtypeStruct((b, r, _K), jnp.int32),
            jax.ShapeDtypeStruct((b, r, _K), jnp.float32),
        ],
    )(q3, mq, mq, mq, mq)

    rk = r * _K
    wts_splat = jnp.broadcast_to(weights.reshape(b, rk, 1), (b, rk, _L))
    read_vectors = pl.kernel(
        _sc_read_body,
        out_type=jax.ShapeDtypeStruct((b, r, w), jnp.float32),
        mesh=plsc.VectorSubcoreMesh(core_axis_name="c", subcore_axis_name="s"),
        scratch_types=[
            pltpu.VMEM((rk,), jnp.int32),
            pltpu.VMEM((rk, _L), jnp.float32),
            pltpu.VMEM((rk, w), jnp.float32),
            pltpu.VMEM((r, w), jnp.float32),
            pltpu.SemaphoreType.DMA,
        ],
    )(memory.reshape(b * m, w),
      read_positions.reshape(b, rk),
      wts_splat)

    return read_vectors, read_positions


# speedup vs baseline: 1.0415x; 1.0415x over previous
import jax
import jax.numpy as jnp
from jax import lax
from jax.experimental import pallas as pl
from jax.experimental.pallas import tpu as pltpu
from jax.experimental.pallas import tpu_sc as plsc

_K = 8
_LW = 128
_NC = 2
_NS = 16
_L = 16


def _proj_body(x_ref, wq_ref, bq_ref, q_ref):
    q = lax.dot_general(
        x_ref[...], wq_ref[...],
        (((1,), (1,)), ((), ())),
        preferred_element_type=jnp.float32,
    )
    q_ref[...] = q + bq_ref[...]


def _topk_body(q_ref, m0_ref, m1_ref, m2_ref, m3_ref, pos_ref, wts_ref):
    q = q_ref[0]
    r_sz = q.shape[0]

    qn = q / (jnp.sqrt(jnp.sum(q * q, axis=-1, keepdims=True)) + 1e-8)

    sims = []
    m_sz = 0
    for ref in (m0_ref, m1_ref, m2_ref, m3_ref):
        mem = ref[0, 0]
        qm, w_dim = mem.shape
        m_sz += qm
        mem3 = mem.reshape(qm // _LW, _LW, w_dim)
        ssq = jnp.sum(mem3 * mem3, axis=-1)
        inv = (1.0 / (jnp.sqrt(ssq) + 1e-8)).reshape(qm, 1)
        mn = mem * inv
        sims.append(lax.dot_general(
            qn, mn,
            (((1,), (1,)), ((), ())),
            preferred_element_type=jnp.float32,
        ))

    q_len = sims[0].shape[1]
    n_ch = q_len // _LW
    iota_l = lax.broadcasted_iota(jnp.int32, (r_sz, _LW), 1)
    neg = jnp.float32(-3.0e38)
    big = jnp.int32(m_sz)

    g1 = sims[0][:, 0:_LW]
    i1 = iota_l
    g2 = jnp.full((r_sz, _LW), neg, jnp.float32)
    i2 = jnp.full((r_sz, _LW), big, jnp.int32)
    for p, s in enumerate(sims):
        for j in range(n_ch):
            if p == 0 and j == 0:
                continue
            v = s[:, j * _LW:(j + 1) * _LW]
            col = iota_l + (p * q_len + j * _LW)
            c1 = v > g1
            c2 = v > g2
            g2 = jnp.where(c1, g1, jnp.where(c2, v, g2))
            i2 = jnp.where(c1, i1, jnp.where(c2, col, i2))
            g1 = jnp.where(c1, v, g1)
            i1 = jnp.where(c1, col, i1)

    def _rescan(excluded):
        rg1 = jnp.full((r_sz, _LW), neg, jnp.float32)
        ri1 = jnp.full((r_sz, _LW), big, jnp.int32)
        rg2 = jnp.full((r_sz, _LW), neg, jnp.float32)
        ri2 = jnp.full((r_sz, _LW), big, jnp.int32)
        for pp, ss in enumerate(sims):
            for jj in range(n_ch):
                vv = ss[:, jj * _LW:(jj + 1) * _LW]
                ccol = iota_l + (pp * q_len + jj * _LW)
                excl = jnp.zeros((r_sz, _LW), jnp.bool_)
                for e in excluded:
                    excl = excl | (ccol == e)
                vv = jnp.where(excl, neg, vv)
                cc1 = vv > rg1
                cc2 = vv > rg2
                rg2 = jnp.where(cc1, rg1, jnp.where(cc2, vv, rg2))
                ri2 = jnp.where(cc1, ri1, jnp.where(cc2, ccol, ri2))
                rg1 = jnp.where(cc1, vv, rg1)
                ri1 = jnp.where(cc1, ccol, ri1)
        return rg1, ri1, rg2, ri2

    denom = jnp.zeros((r_sz, 1), dtype=jnp.float32)
    v0 = None
    cols, ws = [], []
    for k in range(_K):
        vmax = jnp.max(g1, axis=1, keepdims=True)
        csel = jnp.min(jnp.where(g1 == vmax, i1, big),
                       axis=1, keepdims=True)
        cols.append(csel)
        if v0 is None:
            v0 = vmax
        w = jnp.exp(vmax - v0)
        ws.append(w)
        denom = denom + w
        hit = i1 == csel
        third = hit & (i2 == big)
        g1 = jnp.where(hit, g2, g1)
        i1 = jnp.where(hit, i2, i1)
        g2 = jnp.where(hit, neg, g2)
        i2 = jnp.where(hit, big, i2)
        if k < _K - 1:
            need = jnp.any(third)
            g1, i1, g2, i2 = lax.cond(
                need,
                lambda c=tuple(cols): _rescan(c),
                lambda: (g1, i1, g2, i2),
            )

    pos_ref[0] = jnp.concatenate(cols, axis=1)
    wts_ref[0] = jnp.concatenate(ws, axis=1) / denom


def _sc_read_body(mem_ref, pos_ref, wts_ref, out_ref,
                  idxv, wvx, rows, acc, sem):
    cid = lax.axis_index("c")
    sid = lax.axis_index("s")
    b = sid * _NC + cid
    m_sz = mem_ref.shape[0] // (_NC * _NS)
    rk = idxv.shape[0]

    pltpu.sync_copy(pos_ref.at[b], idxv)
    pltpu.sync_copy(wts_ref.at[b], wvx)
    for j in range(rk // _L):
        sl = pl.ds(j * _L, _L)
        idxv[sl] = idxv[sl] + b * m_sz
    pltpu.async_copy(mem_ref.at[idxv], rows, sem).wait()

    r_sz, w_sz = acc.shape
    for r in range(r_sz):
        accs = [jnp.zeros((_L,), jnp.float32) for _ in range(w_sz // _L)]
        for k in range(_K):
            wspl = wvx[r * _K + k]
            for c in range(w_sz // _L):
                accs[c] = accs[c] + wspl * rows[r * _K + k, pl.ds(c * _L, _L)]
        for c in range(w_sz // _L):
            acc[r, pl.ds(c * _L, _L)] = accs[c]
    pltpu.sync_copy(acc, out_ref.at[b])


def kernel(x, memory, W_q, b_q):
    b, m, w = memory.shape
    wr = W_q.shape[0]
    r = wr // w

    q = pl.pallas_call(
        _proj_body,
        out_shape=jax.ShapeDtypeStruct((b, wr), jnp.float32),
    )(x, W_q, b_q.reshape(1, wr))
    q3 = q.reshape(b, r, w)

    n_q = 4
    mq = memory.reshape(b, n_q, m // n_q, w)
    read_positions, weights = pl.pallas_call(
        _topk_body,
        grid=(b,),
        in_specs=[
            pl.BlockSpec((1, r, w), lambda i: (i, 0, 0)),
        ] + [
            pl.BlockSpec((1, 1, m // n_q, w), lambda i, p=p: (i, p, 0, 0))
            for p in range(n_q)
        ],
        out_specs=[
            pl.BlockSpec((1, r, _K), lambda i: (i, 0, 0)),
            pl.BlockSpec((1, r, _K), lambda i: (i, 0, 0)),
        ],
        out_shape=[
            jax.ShapeDtypeStruct((b, r, _K), jnp.int32),
            jax.ShapeDtypeStruct((b, r, _K), jnp.float32),
        ],
    )(q3, mq, mq, mq, mq)

    rk = r * _K
    wts_splat = jnp.broadcast_to(weights.reshape(b, rk, 1), (b, rk, _L))
    read_vectors = pl.kernel(
        _sc_read_body,
        out_type=jax.ShapeDtypeStruct((b, r, w), jnp.float32),
        mesh=plsc.VectorSubcoreMesh(core_axis_name="c", subcore_axis_name="s"),
        scratch_types=[
            pltpu.VMEM((rk,), jnp.int32),
            pltpu.VMEM((rk, _L), jnp.float32),
            pltpu.VMEM((rk, w), jnp.float32),
            pltpu.VMEM((r, w), jnp.float32),
            pltpu.SemaphoreType.DMA,
        ],
    )(memory.reshape(b * m, w),
      read_positions.reshape(b, rk),
      wts_splat)

    return read_vectors, read_positions


# speedup vs baseline: 1.0478x; 1.0061x over previous
import jax
import jax.numpy as jnp
from jax import lax
from jax.experimental import pallas as pl
from jax.experimental.pallas import tpu as pltpu
from jax.experimental.pallas import tpu_sc as plsc

_K = 8
_LW = 128
_NC = 2
_NS = 16
_L = 16


def _proj_body(x_ref, wq_ref, bq_ref, q_ref):
    q = lax.dot_general(
        x_ref[...], wq_ref[...],
        (((1,), (1,)), ((), ())),
        preferred_element_type=jnp.float32,
    )
    q_ref[...] = q + bq_ref[...]


def _topk_body(q_ref, m0_ref, m1_ref, m2_ref, m3_ref, pos_ref, wts_ref):
    for bb in range(q_ref.shape[0]):
        _topk_one(bb, q_ref, m0_ref, m1_ref, m2_ref, m3_ref, pos_ref, wts_ref)


def _topk_one(bb, q_ref, m0_ref, m1_ref, m2_ref, m3_ref, pos_ref, wts_ref):
    q = q_ref[bb]
    r_sz = q.shape[0]

    qn = q / (jnp.sqrt(jnp.sum(q * q, axis=-1, keepdims=True)) + 1e-8)

    sims = []
    m_sz = 0
    for ref in (m0_ref, m1_ref, m2_ref, m3_ref):
        mem = ref[bb, 0]
        qm, w_dim = mem.shape
        m_sz += qm
        mem3 = mem.reshape(qm // _LW, _LW, w_dim)
        ssq = jnp.sum(mem3 * mem3, axis=-1)
        inv = (1.0 / (jnp.sqrt(ssq) + 1e-8)).reshape(qm, 1)
        mn = mem * inv
        sims.append(lax.dot_general(
            qn, mn,
            (((1,), (1,)), ((), ())),
            preferred_element_type=jnp.float32,
        ))

    q_len = sims[0].shape[1]
    n_ch = q_len // _LW
    iota_l = lax.broadcasted_iota(jnp.int32, (r_sz, _LW), 1)
    neg = jnp.float32(-3.0e38)
    big = jnp.int32(m_sz)

    g1 = sims[0][:, 0:_LW]
    i1 = iota_l
    g2 = jnp.full((r_sz, _LW), neg, jnp.float32)
    i2 = jnp.full((r_sz, _LW), big, jnp.int32)
    for p, s in enumerate(sims):
        for j in range(n_ch):
            if p == 0 and j == 0:
                continue
            v = s[:, j * _LW:(j + 1) * _LW]
            col = iota_l + (p * q_len + j * _LW)
            c1 = v > g1
            c2 = v > g2
            g2 = jnp.where(c1, g1, jnp.where(c2, v, g2))
            i2 = jnp.where(c1, i1, jnp.where(c2, col, i2))
            g1 = jnp.where(c1, v, g1)
            i1 = jnp.where(c1, col, i1)

    def _rescan(excluded):
        rg1 = jnp.full((r_sz, _LW), neg, jnp.float32)
        ri1 = jnp.full((r_sz, _LW), big, jnp.int32)
        rg2 = jnp.full((r_sz, _LW), neg, jnp.float32)
        ri2 = jnp.full((r_sz, _LW), big, jnp.int32)
        for pp, ss in enumerate(sims):
            for jj in range(n_ch):
                vv = ss[:, jj * _LW:(jj + 1) * _LW]
                ccol = iota_l + (pp * q_len + jj * _LW)
                excl = jnp.zeros((r_sz, _LW), jnp.bool_)
                for e in excluded:
                    excl = excl | (ccol == e)
                vv = jnp.where(excl, neg, vv)
                cc1 = vv > rg1
                cc2 = vv > rg2
                rg2 = jnp.where(cc1, rg1, jnp.where(cc2, vv, rg2))
                ri2 = jnp.where(cc1, ri1, jnp.where(cc2, ccol, ri2))
                rg1 = jnp.where(cc1, vv, rg1)
                ri1 = jnp.where(cc1, ccol, ri1)
        return rg1, ri1, rg2, ri2

    denom = jnp.zeros((r_sz, 1), dtype=jnp.float32)
    v0 = None
    cols, ws = [], []
    for k in range(_K):
        vmax = jnp.max(g1, axis=1, keepdims=True)
        csel = jnp.min(jnp.where(g1 == vmax, i1, big),
                       axis=1, keepdims=True)
        cols.append(csel)
        if v0 is None:
            v0 = vmax
        w = jnp.exp(vmax - v0)
        ws.append(w)
        denom = denom + w
        hit = i1 == csel
        third = hit & (i2 == big)
        g1 = jnp.where(hit, g2, g1)
        i1 = jnp.where(hit, i2, i1)
        g2 = jnp.where(hit, neg, g2)
        i2 = jnp.where(hit, big, i2)
        if k < _K - 1:
            need = jnp.any(third)
            g1, i1, g2, i2 = lax.cond(
                need,
                lambda c=tuple(cols): _rescan(c),
                lambda: (g1, i1, g2, i2),
            )

    pos_ref[bb] = jnp.concatenate(cols, axis=1)
    wts_ref[bb] = jnp.concatenate(ws, axis=1) / denom


def _sc_read_body(mem_ref, pos_ref, wts_ref, out_ref,
                  idxv, wvx, rows, acc, sem):
    cid = lax.axis_index("c")
    sid = lax.axis_index("s")
    b = sid * _NC + cid
    m_sz = mem_ref.shape[0] // (_NC * _NS)
    rk = idxv.shape[0]

    pltpu.sync_copy(pos_ref.at[b], idxv)
    pltpu.sync_copy(wts_ref.at[b], wvx)
    for j in range(rk // _L):
        sl = pl.ds(j * _L, _L)
        idxv[sl] = idxv[sl] + b * m_sz
    pltpu.async_copy(mem_ref.at[idxv], rows, sem).wait()

    r_sz, w_sz = acc.shape
    for r in range(r_sz):
        accs = [jnp.zeros((_L,), jnp.float32) for _ in range(w_sz // _L)]
        for k in range(_K):
            wspl = wvx[r * _K + k]
            for c in range(w_sz // _L):
                accs[c] = accs[c] + wspl * rows[r * _K + k, pl.ds(c * _L, _L)]
        for c in range(w_sz // _L):
            acc[r, pl.ds(c * _L, _L)] = accs[c]
    pltpu.sync_copy(acc, out_ref.at[b])


def kernel(x, memory, W_q, b_q):
    b, m, w = memory.shape
    wr = W_q.shape[0]
    r = wr // w

    q = pl.pallas_call(
        _proj_body,
        out_shape=jax.ShapeDtypeStruct((b, wr), jnp.float32),
    )(x, W_q, b_q.reshape(1, wr))
    q3 = q.reshape(b, r, w)

    n_q = 4
    n_b = 2
    mq = memory.reshape(b, n_q, m // n_q, w)
    read_positions, weights = pl.pallas_call(
        _topk_body,
        grid=(b // n_b,),
        in_specs=[
            pl.BlockSpec((n_b, r, w), lambda i: (i, 0, 0)),
        ] + [
            pl.BlockSpec((n_b, 1, m // n_q, w), lambda i, p=p: (i, p, 0, 0))
            for p in range(n_q)
        ],
        out_specs=[
            pl.BlockSpec((n_b, r, _K), lambda i: (i, 0, 0)),
            pl.BlockSpec((n_b, r, _K), lambda i: (i, 0, 0)),
        ],
        out_shape=[
            jax.ShapeDtypeStruct((b, r, _K), jnp.int32),
            jax.ShapeDtypeStruct((b, r, _K), jnp.float32),
        ],
    )(q3, mq, mq, mq, mq)

    rk = r * _K
    wts_splat = jnp.broadcast_to(weights.reshape(b, rk, 1), (b, rk, _L))
    read_vectors = pl.kernel(
        _sc_read_body,
        out_type=jax.ShapeDtypeStruct((b, r, w), jnp.float32),
        mesh=plsc.VectorSubcoreMesh(core_axis_name="c", subcore_axis_name="s"),
        scratch_types=[
            pltpu.VMEM((rk,), jnp.int32),
            pltpu.VMEM((rk, _L), jnp.float32),
            pltpu.VMEM((rk, w), jnp.float32),
            pltpu.VMEM((r, w), jnp.float32),
            pltpu.SemaphoreType.DMA,
        ],
    )(memory.reshape(b * m, w),
      read_positions.reshape(b, rk),
      wts_splat)

    return read_vectors, read_positions


# speedup vs baseline: 1.0616x; 1.0131x over previous
import jax
import jax.numpy as jnp
from jax import lax
from jax.experimental import pallas as pl
from jax.experimental.pallas import tpu as pltpu
from jax.experimental.pallas import tpu_sc as plsc

_K = 8
_LW = 128
_NC = 2
_NS = 16
_L = 16


def _proj_body(x_ref, wq_ref, bq_ref, q_ref):
    q = lax.dot_general(
        x_ref[...], wq_ref[...],
        (((1,), (1,)), ((), ())),
        preferred_element_type=jnp.float32,
    )
    q_ref[...] = q + bq_ref[...]


def _topk_body(q_ref, mem_ref, pos_ref, wts_ref):
    mem = mem_ref[0]
    q = q_ref[0]
    m_sz = mem.shape[0]
    r_sz = q.shape[0]

    qn = q / (jnp.sqrt(jnp.sum(q * q, axis=-1, keepdims=True)) + 1e-8)
    mn = mem / (jnp.sqrt(jnp.sum(mem * mem, axis=-1, keepdims=True)) + 1e-8)

    sim = lax.dot_general(
        qn, mn,
        (((1,), (1,)), ((), ())),
        preferred_element_type=jnp.float32,
    )

    iota_m = lax.broadcasted_iota(jnp.int32, (r_sz, m_sz), 1)
    neg_inf = jnp.float32(-jnp.inf)

    denom = jnp.zeros((r_sz, 1), dtype=jnp.float32)
    v0 = None
    cols, ws = [], []
    for _ in range(_K):
        vmax = jnp.max(sim, axis=1, keepdims=True)
        idx = jnp.min(jnp.where(sim == vmax, iota_m, m_sz),
                      axis=1, keepdims=True)
        cols.append(idx)
        if v0 is None:
            v0 = vmax
        w = jnp.exp(vmax - v0)
        ws.append(w)
        denom = denom + w
        sim = jnp.where(iota_m == idx, neg_inf, sim)

    pos_ref[0] = jnp.concatenate(cols, axis=1)
    wts_ref[0] = jnp.concatenate(ws, axis=1) / denom


def _sc_read_body(mem_ref, pos_ref, wts_ref, out_ref,
                  idxv, wvx, rows, acc, sem):
    cid = lax.axis_index("c")
    sid = lax.axis_index("s")
    b = sid * _NC + cid
    m_sz = mem_ref.shape[0] // (_NC * _NS)
    rk = idxv.shape[0]

    pltpu.sync_copy(pos_ref.at[b], idxv)
    pltpu.sync_copy(wts_ref.at[b], wvx)
    for j in range(rk // _L):
        sl = pl.ds(j * _L, _L)
        idxv[sl] = idxv[sl] + b * m_sz
    pltpu.async_copy(mem_ref.at[idxv], rows, sem).wait()

    r_sz, w_sz = acc.shape
    for r in range(r_sz):
        accs = [jnp.zeros((_L,), jnp.float32) for _ in range(w_sz // _L)]
        for k in range(_K):
            wspl = wvx[r * _K + k]
            for c in range(w_sz // _L):
                accs[c] = accs[c] + wspl * rows[r * _K + k, pl.ds(c * _L, _L)]
        for c in range(w_sz // _L):
            acc[r, pl.ds(c * _L, _L)] = accs[c]
    pltpu.sync_copy(acc, out_ref.at[b])


def kernel(x, memory, W_q, b_q):
    b, m, w = memory.shape
    wr = W_q.shape[0]
    r = wr // w

    q = pl.pallas_call(
        _proj_body,
        out_shape=jax.ShapeDtypeStruct((b, wr), jnp.float32),
    )(x, W_q, b_q.reshape(1, wr))
    q3 = q.reshape(b, r, w)

    read_positions, weights = pl.pallas_call(
        _topk_body,
        grid=(b,),
        in_specs=[
            pl.BlockSpec((1, r, w), lambda i: (i, 0, 0)),
            pl.BlockSpec((1, m, w), lambda i: (i, 0, 0)),
        ],
        out_specs=[
            pl.BlockSpec((1, r, _K), lambda i: (i, 0, 0)),
            pl.BlockSpec((1, r, _K), lambda i: (i, 0, 0)),
        ],
        out_shape=[
            jax.ShapeDtypeStruct((b, r, _K), jnp.int32),
            jax.ShapeDtypeStruct((b, r, _K), jnp.float32),
        ],
    )(q3, memory)

    rk = r * _K
    wts_splat = jnp.broadcast_to(weights.reshape(b, rk, 1), (b, rk, _L))
    read_vectors = pl.kernel(
        _sc_read_body,
        out_type=jax.ShapeDtypeStruct((b, r, w), jnp.float32),
        mesh=plsc.VectorSubcoreMesh(core_axis_name="c", subcore_axis_name="s"),
        scratch_types=[
            pltpu.VMEM((rk,), jnp.int32),
            pltpu.VMEM((rk, _L), jnp.float32),
            pltpu.VMEM((rk, w), jnp.float32),
            pltpu.VMEM((r, w), jnp.float32),
            pltpu.SemaphoreType.DMA,
        ],
    )(memory.reshape(b * m, w),
      read_positions.reshape(b, rk),
      wts_splat)

    return read_vectors, read_positions
